# baseline (speedup 1.0000x reference)
import jax
import jax.numpy as jnp
from jax import lax
from jax.experimental import pallas as pl
from jax.experimental.pallas import tpu as pltpu
from jax.experimental.pallas import tpu_sc as plsc

NB, NV, NF = 16, 10000, 128
NC, NS = 2, 16
NW = NC * NS
RPW = NB * NV // NW
CHUNK = 128
GROUP = 192
CHUNKS = []
for g in range(RPW // GROUP):
    CHUNKS.append((g * GROUP, 128, 0, g, False))
    CHUNKS.append((g * GROUP + 128, 64, 128, g, True))
CHUNKS.append((RPW - RPW % GROUP, RPW % GROUP, 0, RPW // GROUP, True))
NCHUNK = len(CHUNKS)
NGRP = RPW // GROUP + 1
GSIZE = [192] * (NGRP - 1) + [RPW % GROUP]
NIDX = 5008
NBUF = 4
DEPTH = 6
HEAD = DEPTH * CHUNK


def _gather_body(y_hbm, rs_hbm, out_hbm, idx_v, bufs, gsems, wsems):
    cid = lax.axis_index("c")
    sid = lax.axis_index("s")
    wid = sid * NC + cid
    b = wid // 2
    h = wid % 2
    ibase = h * RPW
    obase = wid * RPW
    boff = b * NV

    def add_offsets(k0, k1):
        def addk(k, carry):
            sl = pl.ds(pl.multiple_of(k * 16, 16), 16)
            idx_v[sl] = idx_v[sl] + boff
            return carry
        lax.fori_loop(k0, k1, addk, 0)

    def gstart(c):
        start, size, boff_, gi, _last = CHUNKS[c]
        return pltpu.async_copy(
            y_hbm.at[idx_v.at[pl.ds(start, size)]],
            bufs[gi % NBUF].at[pl.ds(boff_, size)],
            gsems[gi % NBUF])

    pltpu.sync_copy(rs_hbm.at[pl.ds(ibase, HEAD)], idx_v.at[pl.ds(0, HEAD)])
    add_offsets(0, HEAD // 16)
    handles = {}
    for c in range(DEPTH):
        handles[("g", c)] = gstart(c)

    pltpu.sync_copy(rs_hbm.at[pl.ds(ibase + HEAD, RPW - HEAD)],
                    idx_v.at[pl.ds(HEAD, RPW - HEAD)])
    add_offsets(HEAD // 16, NIDX // 16)

    for c in range(NCHUNK):
        start, size, boff_, gi, last = CHUNKS[c]
        handles[("g", c)].wait()
        if last:
            handles[("w", gi)] = pltpu.async_copy(
                bufs[gi % NBUF].at[pl.ds(0, GSIZE[gi])],
                out_hbm.at[pl.ds(obase + gi * GROUP, GSIZE[gi])],
                wsems[gi % NBUF])
        n = c + DEPTH
        if n < NCHUNK:
            gi_n = CHUNKS[n][3]
            first_of_group = CHUNKS[n][2] == 0
            if gi_n - NBUF >= 0 and first_of_group:
                handles[("w", gi_n - NBUF)].wait()
            handles[("g", n)] = gstart(n)
    for p in range(NGRP - NBUF, NGRP):
        handles[("w", p)].wait()


def kernel(y, rs):
    y_flat = y.reshape(NB * NV, NF)
    rs = rs.astype(jnp.int32)
    mesh = plsc.VectorSubcoreMesh(core_axis_name="c", subcore_axis_name="s")
    out_flat = pl.kernel(
        _gather_body,
        mesh=mesh,
        out_type=jax.ShapeDtypeStruct((NB * NV, NF), jnp.float32),
        scratch_types=[
            pltpu.VMEM((NIDX,), jnp.int32),
            [pltpu.VMEM((GROUP, NF), jnp.float32) for _ in range(NBUF)],
            [pltpu.SemaphoreType.DMA for _ in range(NBUF)],
            [pltpu.SemaphoreType.DMA for _ in range(NBUF)],
        ],
    )(y_flat, rs)
    return out_flat.reshape(NB, NV, NF)


# speedup vs baseline: 1.0027x; 1.0027x over previous
import jax
import jax.numpy as jnp
from jax import lax
from jax.experimental import pallas as pl
from jax.experimental.pallas import tpu as pltpu
from jax.experimental.pallas import tpu_sc as plsc

NB, NV, NF = 16, 10000, 128
NC, NS = 2, 16
NW = NC * NS
RPW = NB * NV // NW
CHUNK = 128
GROUP = 2 * CHUNK
CHUNKS = []
for g in range(RPW // GROUP):
    CHUNKS.append((g * GROUP, CHUNK, False))
    CHUNKS.append((g * GROUP + CHUNK, CHUNK, True))
CHUNKS.append((RPW - RPW % GROUP, CHUNK, False))
CHUNKS.append((RPW - RPW % GROUP + CHUNK, RPW % GROUP - CHUNK, True))
NCHUNK = len(CHUNKS)
NGRP = NCHUNK // 2
NIDX = 5008
NBUF = 3
DEPTH = 4
HEAD = DEPTH * CHUNK


def _gather_body(y_hbm, rs_hbm, out_hbm, idx_v, bufs, gsems, wsems):
    cid = lax.axis_index("c")
    sid = lax.axis_index("s")
    wid = sid * NC + cid
    b = wid // 2
    h = wid % 2
    ibase = h * RPW
    obase = wid * RPW
    boff = b * NV

    def add_offsets(k0, k1):
        def addk(k, carry):
            sl = pl.ds(pl.multiple_of(k * 16, 16), 16)
            idx_v[sl] = idx_v[sl] + boff
            return carry
        lax.fori_loop(k0, k1, addk, 0)

    def gstart(c):
        start, size, second = CHUNKS[c]
        gi = c // 2
        return pltpu.async_copy(
            y_hbm.at[idx_v.at[pl.ds(start, size)]],
            bufs[gi % NBUF].at[pl.ds(CHUNK if second else 0, size)],
            gsems[gi % NBUF])

    pltpu.sync_copy(rs_hbm.at[pl.ds(ibase, HEAD)], idx_v.at[pl.ds(0, HEAD)])
    add_offsets(0, HEAD // 16)
    handles = {}
    for c in range(DEPTH):
        handles[("g", c)] = gstart(c)

    pltpu.sync_copy(rs_hbm.at[pl.ds(ibase + HEAD, RPW - HEAD)],
                    idx_v.at[pl.ds(HEAD, RPW - HEAD)])
    add_offsets(HEAD // 16, NIDX // 16)

    for c in range(NCHUNK):
        start, size, second = CHUNKS[c]
        gi = c // 2
        handles[("g", c)].wait()
        if second:
            gsize = CHUNK + size
            handles[("w", gi)] = pltpu.async_copy(
                bufs[gi % NBUF].at[pl.ds(0, gsize)],
                out_hbm.at[pl.ds(obase + gi * GROUP, gsize)],
                wsems[gi % NBUF])
        n = c + DEPTH
        if n < NCHUNK:
            gn = CHUNKS[n]
            gi_n = n // 2
            if gi_n - NBUF >= 0 and (not gn[2]):
                handles[("w", gi_n - NBUF)].wait()
            handles[("g", n)] = gstart(n)
    for p in range(NGRP - NBUF, NGRP):
        handles[("w", p)].wait()


def kernel(y, rs):
    y_flat = y.reshape(NB * NV, NF)
    rs = rs.astype(jnp.int32)
    mesh = plsc.VectorSubcoreMesh(core_axis_name="c", subcore_axis_name="s")
    out_flat = pl.kernel(
        _gather_body,
        mesh=mesh,
        out_type=jax.ShapeDtypeStruct((NB * NV, NF), jnp.float32),
        scratch_types=[
            pltpu.VMEM((NIDX,), jnp.int32),
            [pltpu.VMEM((GROUP, NF), jnp.float32) for _ in range(NBUF)],
            [pltpu.SemaphoreType.DMA for _ in range(NBUF)],
            [pltpu.SemaphoreType.DMA for _ in range(NBUF)],
        ],
    )(y_flat, rs)
    return out_flat.reshape(NB, NV, NF)
